# baseline (speedup 1.0000x reference)
import functools

import jax
import jax.numpy as jnp
from jax import lax
from jax.experimental import pallas as pl
from jax.experimental.pallas import tpu as pltpu
from jax.experimental.pallas import tpu_sc as plsc

BATCH = 16384
D = 64
NC = 2
NS = 16
NW = NC * NS
B_PER_W = BATCH // NW


CHUNK = 256
N_ROWS = 100000
PACK_BLOCK = 4000


def _pack_body(u_ref, m_ref, o_ref):
  o_ref[...] = jnp.concatenate([u_ref[...], m_ref[...]], axis=1)


def _tc_pack(user_emb, movie_emb):
  grid = (N_ROWS // PACK_BLOCK,)
  return pl.pallas_call(
      _pack_body,
      grid=grid,
      in_specs=[
          pl.BlockSpec((PACK_BLOCK, D), lambda i: (i, 0)),
          pl.BlockSpec((PACK_BLOCK, D), lambda i: (i, 0)),
      ],
      out_specs=pl.BlockSpec((PACK_BLOCK, 2 * D), lambda i: (i, 0)),
      out_shape=jax.ShapeDtypeStruct((N_ROWS, 2 * D), jnp.float32),
  )(user_emb, movie_emb)


def _sc_gather(big_table, users, movies):
  mesh = plsc.VectorSubcoreMesh(core_axis_name="c", subcore_axis_name="s")

  @functools.partial(
      pl.kernel,
      mesh=mesh,
      out_type=[
          jax.ShapeDtypeStruct((BATCH, 2 * D), jnp.float32),
          jax.ShapeDtypeStruct((BATCH, 2 * D), jnp.float32),
      ],
      scratch_types=[
          pltpu.VMEM((CHUNK,), jnp.int32),
          pltpu.VMEM((CHUNK, 2 * D), jnp.float32),
          pltpu.VMEM((CHUNK,), jnp.int32),
          pltpu.VMEM((CHUNK, 2 * D), jnp.float32),
          pltpu.SemaphoreType.DMA,
          pltpu.SemaphoreType.DMA,
      ],
  )
  def gather_kernel(table_hbm, users_hbm, movies_hbm, ou_hbm, om_hbm,
                    uidx_v, urows_v, midx_v, mrows_v, usem, msem):
    wid = lax.axis_index("s") * NC + lax.axis_index("c")

    @pl.loop(0, B_PER_W // CHUNK)
    def _(i):
      base = wid * B_PER_W + i * CHUNK
      pltpu.sync_copy(users_hbm.at[pl.ds(base, CHUNK)], uidx_v)
      pltpu.sync_copy(movies_hbm.at[pl.ds(base, CHUNK)], midx_v)
      cu = pltpu.async_copy(table_hbm.at[uidx_v], urows_v, usem)
      cm = pltpu.async_copy(table_hbm.at[midx_v], mrows_v, msem)
      cu.wait()
      cm.wait()
      pltpu.sync_copy(urows_v, ou_hbm.at[pl.ds(base, CHUNK)])
      pltpu.sync_copy(mrows_v, om_hbm.at[pl.ds(base, CHUNK)])

  return gather_kernel(big_table, users, movies)


def _mlp_body(u_ref, m_ref, w1a_ref, w1b_ref, w1c_ref, b1_ref, w2_ref, b2_ref,
              o_ref):
  u = u_ref[:, :D]
  m = m_ref[:, D:]
  h = (
      jnp.dot(u * m, w1a_ref[...], preferred_element_type=jnp.float32)
      + jnp.dot(u, w1b_ref[...], preferred_element_type=jnp.float32)
      + jnp.dot(m, w1c_ref[...], preferred_element_type=jnp.float32)
      + b1_ref[...]
  )
  h = jnp.maximum(h, 0.0)
  y = jnp.dot(h, w2_ref[...], preferred_element_type=jnp.float32) + b2_ref[...]
  o_ref[...] = jax.nn.sigmoid(y)


def _tc_mlp(u_g, m_g, W1, b1, W2, b2, block=2048):
  w1t = W1.T
  w1a = w1t[:D]
  w1b = w1t[D:2 * D]
  w1c = w1t[2 * D:]
  b1r = b1.reshape(1, 8)
  w2r = W2.reshape(8, 1)
  b2r = b2.reshape(1, 1)
  grid = (BATCH // block,)
  return pl.pallas_call(
      _mlp_body,
      grid=grid,
      in_specs=[
          pl.BlockSpec((block, 2 * D), lambda i: (i, 0)),
          pl.BlockSpec((block, 2 * D), lambda i: (i, 0)),
          pl.BlockSpec((D, 8), lambda i: (0, 0)),
          pl.BlockSpec((D, 8), lambda i: (0, 0)),
          pl.BlockSpec((D, 8), lambda i: (0, 0)),
          pl.BlockSpec((1, 8), lambda i: (0, 0)),
          pl.BlockSpec((8, 1), lambda i: (0, 0)),
          pl.BlockSpec((1, 1), lambda i: (0, 0)),
      ],
      out_specs=pl.BlockSpec((block, 1), lambda i: (i, 0)),
      out_shape=jax.ShapeDtypeStruct((BATCH, 1), jnp.float32),
  )(u_g, m_g, w1a, w1b, w1c, b1r, w2r, b2r)


@jax.jit
def kernel(users, movies, user_emb, movie_emb, W1, b1, W2, b2):
  users = users.astype(jnp.int32)
  movies = movies.astype(jnp.int32)
  big_table = _tc_pack(user_emb, movie_emb)
  u_g, m_g = _sc_gather(big_table, users, movies)
  return _tc_mlp(u_g, m_g, W1, b1, W2, b2)


# speedup vs baseline: 1.2099x; 1.2099x over previous
import functools

import jax
import jax.numpy as jnp
from jax import lax
from jax.experimental import pallas as pl
from jax.experimental.pallas import tpu as pltpu
from jax.experimental.pallas import tpu_sc as plsc

BATCH = 16384
D = 64
NC = 2
NS = 16
NW = NC * NS
B_PER_W = BATCH // NW


CHUNK = 256
N_ROWS = 100000
PACK_BLOCK = 4000


def _pack_body(u_ref, m_ref, o_ref):
  o_ref[...] = jnp.concatenate([u_ref[...], m_ref[...]], axis=1)


def _tc_pack(user_emb, movie_emb):
  grid = (N_ROWS // PACK_BLOCK,)
  return pl.pallas_call(
      _pack_body,
      grid=grid,
      in_specs=[
          pl.BlockSpec((PACK_BLOCK, D), lambda i: (i, 0)),
          pl.BlockSpec((PACK_BLOCK, D), lambda i: (i, 0)),
      ],
      out_specs=pl.BlockSpec((PACK_BLOCK, 2 * D), lambda i: (i, 0)),
      out_shape=jax.ShapeDtypeStruct((N_ROWS, 2 * D), jnp.float32),
  )(user_emb, movie_emb)


def _sc_gather(big_table, users, movies, nbatch):
  mesh = plsc.VectorSubcoreMesh(core_axis_name="c", subcore_axis_name="s")
  b_per_w = nbatch // NW

  @functools.partial(
      pl.kernel,
      mesh=mesh,
      out_type=[
          jax.ShapeDtypeStruct((nbatch, 2 * D), jnp.float32),
          jax.ShapeDtypeStruct((nbatch, 2 * D), jnp.float32),
      ],
      scratch_types=[
          pltpu.VMEM((CHUNK,), jnp.int32),
          pltpu.VMEM((CHUNK, 2 * D), jnp.float32),
          pltpu.VMEM((CHUNK,), jnp.int32),
          pltpu.VMEM((CHUNK, 2 * D), jnp.float32),
          pltpu.SemaphoreType.DMA,
          pltpu.SemaphoreType.DMA,
      ],
  )
  def gather_kernel(table_hbm, users_hbm, movies_hbm, ou_hbm, om_hbm,
                    uidx_v, urows_v, midx_v, mrows_v, usem, msem):
    wid = lax.axis_index("s") * NC + lax.axis_index("c")

    @pl.loop(0, b_per_w // CHUNK)
    def _(i):
      base = wid * b_per_w + i * CHUNK
      pltpu.sync_copy(users_hbm.at[pl.ds(base, CHUNK)], uidx_v)
      pltpu.sync_copy(movies_hbm.at[pl.ds(base, CHUNK)], midx_v)
      cu = pltpu.async_copy(table_hbm.at[uidx_v], urows_v, usem)
      cm = pltpu.async_copy(table_hbm.at[midx_v], mrows_v, msem)
      cu.wait()
      cm.wait()
      pltpu.sync_copy(urows_v, ou_hbm.at[pl.ds(base, CHUNK)])
      pltpu.sync_copy(mrows_v, om_hbm.at[pl.ds(base, CHUNK)])

  return gather_kernel(big_table, users, movies)


def _mlp_body(u_ref, m_ref, w1a_ref, w1b_ref, w1c_ref, b1_ref, w2_ref, b2_ref,
              o_ref):
  u = u_ref[:, :D]
  m = m_ref[:, D:]
  h = (
      jnp.dot(u * m, w1a_ref[...], preferred_element_type=jnp.float32)
      + jnp.dot(u, w1b_ref[...], preferred_element_type=jnp.float32)
      + jnp.dot(m, w1c_ref[...], preferred_element_type=jnp.float32)
      + b1_ref[...]
  )
  h = jnp.maximum(h, 0.0)
  y = jnp.dot(h, w2_ref[...], preferred_element_type=jnp.float32) + b2_ref[...]
  o_ref[...] = jax.nn.sigmoid(y)


def _tc_mlp(u_g, m_g, W1, b1, W2, b2, nbatch, block=2048):
  w1t = W1.T
  w1a = w1t[:D]
  w1b = w1t[D:2 * D]
  w1c = w1t[2 * D:]
  b1r = b1.reshape(1, 8)
  w2r = W2.reshape(8, 1)
  b2r = b2.reshape(1, 1)
  grid = (nbatch // block,)
  return pl.pallas_call(
      _mlp_body,
      grid=grid,
      in_specs=[
          pl.BlockSpec((block, 2 * D), lambda i: (i, 0)),
          pl.BlockSpec((block, 2 * D), lambda i: (i, 0)),
          pl.BlockSpec((D, 8), lambda i: (0, 0)),
          pl.BlockSpec((D, 8), lambda i: (0, 0)),
          pl.BlockSpec((D, 8), lambda i: (0, 0)),
          pl.BlockSpec((1, 8), lambda i: (0, 0)),
          pl.BlockSpec((8, 1), lambda i: (0, 0)),
          pl.BlockSpec((1, 1), lambda i: (0, 0)),
      ],
      out_specs=pl.BlockSpec((block, 1), lambda i: (i, 0)),
      out_shape=jax.ShapeDtypeStruct((nbatch, 1), jnp.float32),
  )(u_g, m_g, w1a, w1b, w1c, b1r, w2r, b2r)


@jax.jit
def kernel(users, movies, user_emb, movie_emb, W1, b1, W2, b2):
  users = users.astype(jnp.int32)
  movies = movies.astype(jnp.int32)
  big_table = jnp.concatenate([user_emb, movie_emb], axis=1)
  npipe = 2
  piece = BATCH // npipe
  outs = []
  for p in range(npipe):
    sl = slice(p * piece, (p + 1) * piece)
    u_g, m_g = _sc_gather(big_table, users[sl], movies[sl], piece)
    outs.append(_tc_mlp(u_g, m_g, W1, b1, W2, b2, piece))
  return jnp.concatenate(outs, axis=0)
